# Initial kernel scaffold; baseline (speedup 1.0000x reference)
import jax
import jax.numpy as jnp
from jax.experimental import pallas as pl


def kernel(x, Wg, bias, Wsh_up, Wsh_down, W_up, W_down):
    raise NotImplementedError("write your pallas kernel here")


# speedup vs baseline: 5.3869x; 5.3869x over previous
import functools

import jax
import jax.numpy as jnp
from jax.experimental import pallas as pl
from jax.experimental.pallas import tpu as pltpu

N_TOK = 4096
D = 768
E = 64
EH = 256
SH = 512
TBLK = 512
N_TBLK = N_TOK // TBLK


def _silu(v):
    return v * jax.nn.sigmoid(v)


def _gate_body(x_ref, wg_ref, b_ref, e0_ref, e1_ref, w0_ref, w1_ref):
    x = x_ref[...].astype(jnp.bfloat16)
    logits = jax.lax.dot_general(
        x, wg_ref[...].astype(jnp.bfloat16), (((1,), (1,)), ((), ())),
        preferred_element_type=jnp.float32)
    scores = jax.nn.sigmoid(logits)
    b = scores + b_ref[...]
    iota = jax.lax.broadcasted_iota(jnp.int32, b.shape, 1)
    m1 = jnp.max(b, axis=1, keepdims=True)
    i1 = jnp.min(jnp.where(b == m1, iota, E), axis=1, keepdims=True)
    s1 = jnp.sum(jnp.where(iota == i1, scores, 0.0), axis=1, keepdims=True)
    b2 = jnp.where(iota == i1, -jnp.inf, b)
    m2 = jnp.max(b2, axis=1, keepdims=True)
    i2 = jnp.min(jnp.where(b2 == m2, iota, E), axis=1, keepdims=True)
    s2 = jnp.sum(jnp.where(iota == i2, scores, 0.0), axis=1, keepdims=True)
    tot = s1 + s2
    e0_ref[...] = i1
    e1_ref[...] = i2
    w0_ref[...] = s1 / tot
    w1_ref[...] = s2 / tot


def _gate(x, Wg, bias):
    return pl.pallas_call(
        _gate_body,
        grid=(N_TBLK,),
        in_specs=[
            pl.BlockSpec((TBLK, D), lambda i: (i, 0)),
            pl.BlockSpec((E, D), lambda i: (0, 0)),
            pl.BlockSpec((1, E), lambda i: (0, 0)),
        ],
        out_specs=[
            pl.BlockSpec((TBLK, 1), lambda i: (i, 0)),
            pl.BlockSpec((TBLK, 1), lambda i: (i, 0)),
            pl.BlockSpec((TBLK, 1), lambda i: (i, 0)),
            pl.BlockSpec((TBLK, 1), lambda i: (i, 0)),
        ],
        out_shape=[
            jax.ShapeDtypeStruct((N_TOK, 1), jnp.int32),
            jax.ShapeDtypeStruct((N_TOK, 1), jnp.int32),
            jax.ShapeDtypeStruct((N_TOK, 1), jnp.float32),
            jax.ShapeDtypeStruct((N_TOK, 1), jnp.float32),
        ],
    )(x, Wg, bias.reshape(1, E))


def _dense_moe_body(x_ref, wu_ref, wd_ref, e0_ref, e1_ref, w0_ref, w1_ref,
                    out_ref):
    e = pl.program_id(0)

    @pl.when(e == 0)
    def _():
        out_ref[...] = jnp.zeros_like(out_ref)

    wu = wu_ref[0].astype(jnp.bfloat16)
    wd = wd_ref[0].astype(jnp.bfloat16)
    scale = (jnp.where(e0_ref[...] == e, w0_ref[...], 0.0)
             + jnp.where(e1_ref[...] == e, w1_ref[...], 0.0))
    for t in range(N_TBLK):
        lo, hi = t * TBLK, (t + 1) * TBLK
        xb = x_ref[lo:hi, :].astype(jnp.bfloat16)
        up = jax.lax.dot_general(xb, wu, (((1,), (1,)), ((), ())),
                                 preferred_element_type=jnp.float32)
        h = _silu(up[:, EH:]) * up[:, :EH]
        dn = jax.lax.dot_general(h.astype(jnp.bfloat16), wd,
                                 (((1,), (1,)), ((), ())),
                                 preferred_element_type=jnp.float32)
        out_ref[lo:hi, :] += dn * scale[lo:hi, :]


def _dense_moe(x, W_up, W_down, e0, e1, w0, w1):
    return pl.pallas_call(
        _dense_moe_body,
        grid=(E,),
        in_specs=[
            pl.BlockSpec((N_TOK, D), lambda e: (0, 0)),
            pl.BlockSpec((1, 2 * EH, D), lambda e: (e, 0, 0)),
            pl.BlockSpec((1, D, EH), lambda e: (e, 0, 0)),
            pl.BlockSpec((N_TOK, 1), lambda e: (0, 0)),
            pl.BlockSpec((N_TOK, 1), lambda e: (0, 0)),
            pl.BlockSpec((N_TOK, 1), lambda e: (0, 0)),
            pl.BlockSpec((N_TOK, 1), lambda e: (0, 0)),
        ],
        out_specs=pl.BlockSpec((N_TOK, D), lambda e: (0, 0)),
        out_shape=jax.ShapeDtypeStruct((N_TOK, D), jnp.float32),
    )(x, W_up, W_down, e0, e1, w0, w1)


def _final_body(x_ref, r_ref, wu_ref, wd_ref, out_ref):
    xb = x_ref[...].astype(jnp.bfloat16)
    wu = wu_ref[...].astype(jnp.bfloat16)
    wd = wd_ref[...].astype(jnp.bfloat16)
    up = jax.lax.dot_general(xb, wu, (((1,), (1,)), ((), ())),
                             preferred_element_type=jnp.float32)
    h = _silu(up[:, SH:]) * up[:, :SH]
    sh = jax.lax.dot_general(h.astype(jnp.bfloat16), wd,
                             (((1,), (1,)), ((), ())),
                             preferred_element_type=jnp.float32)
    out_ref[...] = r_ref[...] + sh


def _final(x, routed, Wsh_up, Wsh_down):
    return pl.pallas_call(
        _final_body,
        grid=(N_TBLK,),
        in_specs=[
            pl.BlockSpec((TBLK, D), lambda i: (i, 0)),
            pl.BlockSpec((TBLK, D), lambda i: (i, 0)),
            pl.BlockSpec((2 * SH, D), lambda i: (0, 0)),
            pl.BlockSpec((D, SH), lambda i: (0, 0)),
        ],
        out_specs=pl.BlockSpec((TBLK, D), lambda i: (i, 0)),
        out_shape=jax.ShapeDtypeStruct((N_TOK, D), jnp.float32),
    )(x, routed, Wsh_up, Wsh_down)


def kernel(x, Wg, bias, Wsh_up, Wsh_down, W_up, W_down):
    e0, e1, w0, w1 = _gate(x, Wg, bias)
    routed = _dense_moe(x, W_up, W_down, e0, e1, w0, w1)
    return _final(x, routed, Wsh_up, Wsh_down)
